# Initial kernel scaffold; baseline (speedup 1.0000x reference)
import jax
import jax.numpy as jnp
from jax.experimental import pallas as pl


def kernel(x, edge_index, edge_attr, batch, W0, b0, Wn1, bn1, Wn2, bn2, Wroot, bconv, Wih, Whh, bih, bhh, Wih_s, Whh_s, bih_s, bhh_s, Wout, bout):
    raise NotImplementedError("write your pallas kernel here")


# speedup vs baseline: 1.9361x; 1.9361x over previous
import functools

import jax
import jax.numpy as jnp
import numpy as np
from jax import lax
from jax.experimental import pallas as pl
from jax.experimental.pallas import tpu as pltpu
from jax.experimental.pallas import tpu_sc as plsc

N = 10000
E = 160000
F_IN = 14
DIM = 32
B = 64
OUT_DIM = 16

NC = 2
NS = 16
NW = NC * NS
NP = 10240
EP = 163840
EPW = EP // NW
CHUNK = 128
NCHUNK = EPW // CHUNK
SP = NP // NS

BLK_E = 1024
BLK_N = 1024

_SC_MESH = dict(core_axis_name="c", subcore_axis_name="s")
_SC_PARAMS = pltpu.CompilerParams(use_tc_tiling_on_sc=False)


def _leaky(v):
    return jnp.where(v >= 0, v, 0.01 * v)


def _sc_gather(table, idx):
    @functools.partial(
        pl.kernel,
        out_type=jax.ShapeDtypeStruct((EP, DIM), jnp.float32),
        mesh=plsc.VectorSubcoreMesh(**_SC_MESH),
        compiler_params=_SC_PARAMS,
        scratch_types=[
            pltpu.VMEM((EPW,), jnp.int32),
            pltpu.VMEM((CHUNK, DIM), jnp.float32),
            pltpu.SemaphoreType.DMA,
        ],
    )
    def k(table_hbm, idx_hbm, out_hbm, idx_v, rows_v, sem):
        wid = lax.axis_index("s") * NC + lax.axis_index("c")
        base = wid * EPW
        pltpu.sync_copy(idx_hbm.at[pl.ds(base, EPW)], idx_v)

        @pl.loop(0, NCHUNK)
        def _(j):
            off = j * CHUNK
            pltpu.async_copy(
                table_hbm.at[idx_v.at[pl.ds(off, CHUNK)]], rows_v, sem
            ).wait()
            pltpu.sync_copy(rows_v, out_hbm.at[pl.ds(base + off, CHUNK)])

    return k(table, idx)


def _sc_scatter_add(msg, dst3, zrow):
    @functools.partial(
        pl.kernel,
        out_type=jax.ShapeDtypeStruct((NC, NP, DIM), jnp.float32),
        mesh=plsc.VectorSubcoreMesh(**_SC_MESH),
        compiler_params=_SC_PARAMS,
        scratch_types=[
            pltpu.VMEM((NCHUNK, CHUNK), jnp.int32),
            pltpu.VMEM((CHUNK, DIM), jnp.float32),
            pltpu.VMEM_SHARED((NP, DIM), jnp.float32),
        ],
    )
    def k(msg_hbm, dst_hbm, z_hbm, out_hbm, idx_v, mbuf, acc):
        cid = lax.axis_index("c")
        sid = lax.axis_index("s")
        wid = sid * NC + cid
        pltpu.sync_copy(z_hbm, acc.at[pl.ds(sid * SP, SP)])
        pltpu.sync_copy(dst_hbm.at[wid], idx_v)
        plsc.subcore_barrier()
        base = wid * EPW

        @pl.loop(0, NCHUNK)
        def _(j):
            pltpu.sync_copy(msg_hbm.at[pl.ds(base + j * CHUNK, CHUNK)], mbuf)
            pltpu.sync_copy(mbuf, acc.at[idx_v.at[j]], add=True)

        plsc.subcore_barrier()
        pltpu.sync_copy(
            acc.at[pl.ds(sid * SP, SP)], out_hbm.at[cid, pl.ds(sid * SP, SP)]
        )

    return k(msg, dst3, zrow)


def _row_matmul_act(xp, WT, brow, n_valid, blk, act):
    rows_total, f_in = xp.shape
    f_out = WT.shape[1]

    def body(x_ref, w_ref, b_ref, o_ref):
        i = pl.program_id(0)
        v = jnp.dot(x_ref[...], w_ref[...], preferred_element_type=jnp.float32)
        v = v + b_ref[...]
        if act:
            v = _leaky(v)
        rows = i * blk + lax.broadcasted_iota(jnp.int32, (blk, 1), 0)
        o_ref[...] = jnp.where(rows < n_valid, v, 0.0)

    return pl.pallas_call(
        body,
        grid=(rows_total // blk,),
        in_specs=[
            pl.BlockSpec((blk, f_in), lambda i: (i, 0)),
            pl.BlockSpec((f_in, f_out), lambda i: (0, 0)),
            pl.BlockSpec((1, f_out), lambda i: (0, 0)),
        ],
        out_specs=pl.BlockSpec((blk, f_out), lambda i: (i, 0)),
        out_shape=jax.ShapeDtypeStruct((rows_total, f_out), jnp.float32),
    )(xp, WT, brow)


def _msg_bilinear(xj, hid, Rm, Tm, Mm, Bn2):
    def body(x_ref, h_ref, r_ref, t_ref, m_ref, b_ref, o_ref):
        xb = x_ref[...]
        p = jnp.dot(xb, r_ref[...], preferred_element_type=jnp.float32)
        p = p * jnp.dot(h_ref[...], t_ref[...], preferred_element_type=jnp.float32)
        o_ref[...] = (
            jnp.dot(p, m_ref[...], preferred_element_type=jnp.float32)
            + jnp.dot(xb, b_ref[...], preferred_element_type=jnp.float32)
        )

    d2 = DIM * DIM
    return pl.pallas_call(
        body,
        grid=(EP // BLK_E,),
        in_specs=[
            pl.BlockSpec((BLK_E, DIM), lambda i: (i, 0)),
            pl.BlockSpec((BLK_E, DIM), lambda i: (i, 0)),
            pl.BlockSpec((DIM, d2), lambda i: (0, 0)),
            pl.BlockSpec((DIM, d2), lambda i: (0, 0)),
            pl.BlockSpec((d2, DIM), lambda i: (0, 0)),
            pl.BlockSpec((DIM, DIM), lambda i: (0, 0)),
        ],
        out_specs=pl.BlockSpec((BLK_E, DIM), lambda i: (i, 0)),
        out_shape=jax.ShapeDtypeStruct((EP, DIM), jnp.float32),
    )(xj, hid, Rm, Tm, Mm, Bn2)


def _deg_finalize(degP):
    def body(p_ref, o_ref):
        deg = p_ref[0, :, 0:1] + p_ref[1, :, 0:1]
        o_ref[...] = 1.0 / jnp.maximum(deg, 1.0)

    return pl.pallas_call(
        body,
        grid=(NP // BLK_N,),
        in_specs=[pl.BlockSpec((NC, BLK_N, DIM), lambda i: (0, i, 0))],
        out_specs=pl.BlockSpec((BLK_N, 1), lambda i: (i, 0)),
        out_shape=jax.ShapeDtypeStruct((NP, 1), jnp.float32),
    )(degP)


def _node_update(aggP, inv_deg, h, Wroot, WihT, WhhT, bconv_r, bih_r, bhh_r):
    def body(p_ref, iv_ref, h_ref, wr_ref, wi_ref, wh_ref, bc_ref, bi_ref,
             bh_ref, o_ref):
        i = pl.program_id(0)
        h_ = h_ref[...]
        agg = (p_ref[0] + p_ref[1]) * iv_ref[...]
        m = _leaky(
            agg
            + jnp.dot(h_, wr_ref[...], preferred_element_type=jnp.float32)
            + bc_ref[...]
        )
        gi = jnp.dot(m, wi_ref[...], preferred_element_type=jnp.float32) + bi_ref[...]
        gh = jnp.dot(h_, wh_ref[...], preferred_element_type=jnp.float32) + bh_ref[...]
        r = jax.nn.sigmoid(gi[:, 0:DIM] + gh[:, 0:DIM])
        z = jax.nn.sigmoid(gi[:, DIM:2 * DIM] + gh[:, DIM:2 * DIM])
        n = jnp.tanh(gi[:, 2 * DIM:3 * DIM] + r * gh[:, 2 * DIM:3 * DIM])
        hn = (1.0 - z) * n + z * h_
        rows = i * BLK_N + lax.broadcasted_iota(jnp.int32, (BLK_N, 1), 0)
        o_ref[...] = jnp.where(rows < N, hn, 0.0)

    g3 = 3 * DIM
    return pl.pallas_call(
        body,
        grid=(NP // BLK_N,),
        in_specs=[
            pl.BlockSpec((NC, BLK_N, DIM), lambda i: (0, i, 0)),
            pl.BlockSpec((BLK_N, 1), lambda i: (i, 0)),
            pl.BlockSpec((BLK_N, DIM), lambda i: (i, 0)),
            pl.BlockSpec((DIM, DIM), lambda i: (0, 0)),
            pl.BlockSpec((DIM, g3), lambda i: (0, 0)),
            pl.BlockSpec((DIM, g3), lambda i: (0, 0)),
            pl.BlockSpec((1, DIM), lambda i: (0, 0)),
            pl.BlockSpec((1, g3), lambda i: (0, 0)),
            pl.BlockSpec((1, g3), lambda i: (0, 0)),
        ],
        out_specs=pl.BlockSpec((BLK_N, DIM), lambda i: (i, 0)),
        out_shape=jax.ShapeDtypeStruct((NP, DIM), jnp.float32),
    )(aggP, inv_deg, h, Wroot, WihT, WhhT, bconv_r, bih_r, bhh_r)


def _set2set_out(h, batch_col, bihs_r, bhhs_r, WqT, WrT, bout_r):
    def body(h_ref, b_ref, bi_ref, bh_ref, wq_ref, wr_ref, bo_ref, o_ref):
        h_ = h_ref[...]
        bb = b_ref[...]
        g = bi_ref[...] + bh_ref[...]
        ig = jax.nn.sigmoid(g[:, 0:DIM])
        fg = jax.nn.sigmoid(g[:, DIM:2 * DIM])
        gg = jnp.tanh(g[:, 2 * DIM:3 * DIM])
        og = jax.nn.sigmoid(g[:, 3 * DIM:4 * DIM])
        cs = ig * gg + fg * 0.0
        q_row = og * jnp.tanh(cs)
        e = jnp.sum(h_ * q_row, axis=1, keepdims=True)
        ids = lax.broadcasted_iota(jnp.int32, (NP, B), 1)
        oh = (bb == ids).astype(jnp.float32)
        neg = jnp.float32(-1e30)
        emax_b = jnp.max(jnp.where(oh > 0, e, neg), axis=0, keepdims=True)
        emax_b = jnp.where(emax_b > neg * 0.5, emax_b, 0.0)
        emax_n = jnp.sum(oh * emax_b, axis=1, keepdims=True)
        valid = jnp.sum(oh, axis=1, keepdims=True)
        a_un = jnp.exp(e - emax_n) * valid
        denom_b = lax.dot_general(oh, a_un, (((0,), (0,)), ((), ())),
                                  preferred_element_type=jnp.float32)
        denom_n = jnp.dot(oh, denom_b, preferred_element_type=jnp.float32)
        a = a_un / jnp.where(denom_n > 0, denom_n, 1.0)
        r_vec = lax.dot_general(oh, a * h_, (((0,), (0,)), ((), ())),
                                preferred_element_type=jnp.float32)
        o_ref[...] = (
            jnp.dot(q_row, wq_ref[...], preferred_element_type=jnp.float32)
            + jnp.dot(r_vec, wr_ref[...], preferred_element_type=jnp.float32)
            + bo_ref[...]
        )

    return pl.pallas_call(
        body,
        out_shape=jax.ShapeDtypeStruct((B, OUT_DIM), jnp.float32),
    )(h, batch_col, bihs_r, bhhs_r, WqT, WrT, bout_r)


_R_EXPAND = np.repeat(np.eye(DIM, dtype=np.float32), DIM, axis=1)
_T_EXPAND = np.tile(np.eye(DIM, dtype=np.float32), (1, DIM))


def kernel(x, edge_index, edge_attr, batch, W0, b0, Wn1, bn1, Wn2, bn2,
           Wroot, bconv, Wih, Whh, bih, bhh, Wih_s, Whh_s, bih_s, bhh_s,
           Wout, bout):
    src = edge_index[0]
    dst = edge_index[1]
    pad_idx = jnp.full((EP - E,), N, jnp.int32)
    srcp = jnp.concatenate([src, pad_idx])
    dst3 = jnp.concatenate([dst, pad_idx]).reshape(NW, NCHUNK, CHUNK)

    xp = jnp.pad(x, ((0, NP - N), (0, 0)))
    eap = jnp.pad(edge_attr, ((0, EP - E), (0, 0)))
    batch_col = jnp.pad(batch, (0, NP - N), constant_values=B).reshape(NP, 1)

    Mm = Wn2.reshape(DIM, DIM, DIM).transpose(0, 2, 1).reshape(DIM * DIM, DIM)
    Bn2 = bn2.reshape(DIM, DIM)
    Rm = jnp.asarray(_R_EXPAND)
    Tm = jnp.asarray(_T_EXPAND)
    zrow = jnp.zeros((SP, DIM), jnp.float32)
    onesE = jnp.ones((EP, DIM), jnp.float32)

    h0 = _row_matmul_act(xp, W0.T, b0.reshape(1, DIM), N, BLK_N, True)
    hid = _row_matmul_act(eap, Wn1.T, bn1.reshape(1, DIM), EP, BLK_E, True)

    degP = _sc_scatter_add(onesE, dst3, zrow)
    inv_deg = _deg_finalize(degP)

    WihT = Wih.T
    WhhT = Whh.T
    bconv_r = bconv.reshape(1, DIM)
    bih_r = bih.reshape(1, 3 * DIM)
    bhh_r = bhh.reshape(1, 3 * DIM)

    def step(_, h):
        xj = _sc_gather(h, srcp)
        msg = _msg_bilinear(xj, hid, Rm, Tm, Mm, Bn2)
        aggP = _sc_scatter_add(msg, dst3, zrow)
        return _node_update(aggP, inv_deg, h, Wroot, WihT, WhhT,
                            bconv_r, bih_r, bhh_r)

    h = lax.fori_loop(0, 6, step, h0)

    WoutT = Wout.T
    return _set2set_out(h, batch_col,
                        bih_s.reshape(1, 4 * DIM), bhh_s.reshape(1, 4 * DIM),
                        WoutT[:DIM], WoutT[DIM:], bout.reshape(1, OUT_DIM))


# speedup vs baseline: 2.2363x; 1.1551x over previous
import functools

import jax
import jax.numpy as jnp
import numpy as np
from jax import lax
from jax.experimental import pallas as pl
from jax.experimental.pallas import tpu as pltpu
from jax.experimental.pallas import tpu_sc as plsc

N = 10000
E = 160000
F_IN = 14
DIM = 32
B = 64
OUT_DIM = 16

NC = 2
NS = 16
NW = NC * NS
NP = 10240
EP = 163840
EPW = EP // NW
CHUNK = 128
NCHUNK = EPW // CHUNK
SP = NP // NS

BLK_E = 1024
BLK_N = 1024

_SC_MESH = dict(core_axis_name="c", subcore_axis_name="s")
_SC_PARAMS = pltpu.CompilerParams(use_tc_tiling_on_sc=False)


def _leaky(v):
    return jnp.where(v >= 0, v, 0.01 * v)


def _sc_gather(table, idx):
    @functools.partial(
        pl.kernel,
        out_type=jax.ShapeDtypeStruct((EP, DIM), jnp.float32),
        mesh=plsc.VectorSubcoreMesh(**_SC_MESH),
        compiler_params=_SC_PARAMS,
        scratch_types=[
            pltpu.VMEM((EPW,), jnp.int32),
            pltpu.VMEM((2, CHUNK, DIM), jnp.float32),
            pltpu.SemaphoreType.DMA((2,)),
            pltpu.SemaphoreType.DMA((2,)),
        ],
    )
    def k(table_hbm, idx_hbm, out_hbm, idx_v, rows_v, gsem, osem):
        wid = lax.axis_index("s") * NC + lax.axis_index("c")
        base = wid * EPW
        pltpu.sync_copy(idx_hbm.at[pl.ds(base, EPW)], idx_v)

        def start(j):
            return pltpu.async_copy(
                table_hbm.at[idx_v.at[pl.ds(j * CHUNK, CHUNK)]],
                rows_v.at[j % 2], gsem.at[j % 2],
            )

        gd = [start(0), None]
        od = [None, None]
        for j in range(NCHUNK):
            b = j % 2
            gd[b].wait()
            od[b] = pltpu.async_copy(
                rows_v.at[b], out_hbm.at[pl.ds(base + j * CHUNK, CHUNK)],
                osem.at[b],
            )
            nb = (j + 1) % 2
            if j + 1 < NCHUNK:
                if od[nb] is not None:
                    od[nb].wait()
                gd[nb] = start(j + 1)
        od[(NCHUNK - 1) % 2].wait()
        od[NCHUNK % 2].wait()

    return k(table, idx)


def _sc_scatter_add(msg, dst3, zrow):
    @functools.partial(
        pl.kernel,
        out_type=jax.ShapeDtypeStruct((NC, NP, DIM), jnp.float32),
        mesh=plsc.VectorSubcoreMesh(**_SC_MESH),
        compiler_params=_SC_PARAMS,
        scratch_types=[
            pltpu.VMEM((NCHUNK, CHUNK), jnp.int32),
            pltpu.VMEM((2, CHUNK, DIM), jnp.float32),
            pltpu.VMEM_SHARED((NP, DIM), jnp.float32),
            pltpu.SemaphoreType.DMA((2,)),
        ],
    )
    def k(msg_hbm, dst_hbm, z_hbm, out_hbm, idx_v, mbuf, acc, msem):
        cid = lax.axis_index("c")
        sid = lax.axis_index("s")
        wid = sid * NC + cid
        pltpu.sync_copy(z_hbm, acc.at[pl.ds(sid * SP, SP)])
        pltpu.sync_copy(dst_hbm.at[wid], idx_v)
        plsc.subcore_barrier()
        base = wid * EPW

        def start(j):
            return pltpu.async_copy(
                msg_hbm.at[pl.ds(base + j * CHUNK, CHUNK)],
                mbuf.at[j % 2], msem.at[j % 2],
            )

        md = [start(0), None]
        for j in range(NCHUNK):
            b = j % 2
            if j + 1 < NCHUNK:
                md[(j + 1) % 2] = start(j + 1)
            md[b].wait()
            pltpu.sync_copy(mbuf.at[b], acc.at[idx_v.at[j]], add=True)

        plsc.subcore_barrier()
        pltpu.sync_copy(
            acc.at[pl.ds(sid * SP, SP)], out_hbm.at[cid, pl.ds(sid * SP, SP)]
        )

    return k(msg, dst3, zrow)


def _row_matmul_act(xp, WT, brow, n_valid, blk, act):
    rows_total, f_in = xp.shape
    f_out = WT.shape[1]

    def body(x_ref, w_ref, b_ref, o_ref):
        i = pl.program_id(0)
        v = jnp.dot(x_ref[...], w_ref[...], preferred_element_type=jnp.float32)
        v = v + b_ref[...]
        if act:
            v = _leaky(v)
        rows = i * blk + lax.broadcasted_iota(jnp.int32, (blk, 1), 0)
        o_ref[...] = jnp.where(rows < n_valid, v, 0.0)

    return pl.pallas_call(
        body,
        grid=(rows_total // blk,),
        in_specs=[
            pl.BlockSpec((blk, f_in), lambda i: (i, 0)),
            pl.BlockSpec((f_in, f_out), lambda i: (0, 0)),
            pl.BlockSpec((1, f_out), lambda i: (0, 0)),
        ],
        out_specs=pl.BlockSpec((blk, f_out), lambda i: (i, 0)),
        out_shape=jax.ShapeDtypeStruct((rows_total, f_out), jnp.float32),
    )(xp, WT, brow)


def _msg_bilinear(xj, hid, Rm, Wn2T, Sm, Bn2):
    def body(x_ref, h_ref, r_ref, w_ref, s_ref, b_ref, o_ref):
        xb = x_ref[...]
        y = jnp.dot(h_ref[...], w_ref[...], preferred_element_type=jnp.float32)
        z = jnp.dot(xb, r_ref[...], preferred_element_type=jnp.float32) * y
        for wdt in (512, 256, 128):
            z = z[:, :wdt] + z[:, wdt:2 * wdt]
        o_ref[...] = (
            jnp.dot(z, s_ref[...], preferred_element_type=jnp.float32)
            + jnp.dot(xb, b_ref[...], preferred_element_type=jnp.float32)
        )

    d2 = DIM * DIM
    return pl.pallas_call(
        body,
        grid=(EP // BLK_E,),
        in_specs=[
            pl.BlockSpec((BLK_E, DIM), lambda i: (i, 0)),
            pl.BlockSpec((BLK_E, DIM), lambda i: (i, 0)),
            pl.BlockSpec((DIM, d2), lambda i: (0, 0)),
            pl.BlockSpec((DIM, d2), lambda i: (0, 0)),
            pl.BlockSpec((128, DIM), lambda i: (0, 0)),
            pl.BlockSpec((DIM, DIM), lambda i: (0, 0)),
        ],
        out_specs=pl.BlockSpec((BLK_E, DIM), lambda i: (i, 0)),
        out_shape=jax.ShapeDtypeStruct((EP, DIM), jnp.float32),
    )(xj, hid, Rm, Wn2T, Sm, Bn2)


def _deg_finalize(degP):
    def body(p_ref, o_ref):
        deg = p_ref[0, :, 0:1] + p_ref[1, :, 0:1]
        o_ref[...] = 1.0 / jnp.maximum(deg, 1.0)

    return pl.pallas_call(
        body,
        grid=(NP // BLK_N,),
        in_specs=[pl.BlockSpec((NC, BLK_N, DIM), lambda i: (0, i, 0))],
        out_specs=pl.BlockSpec((BLK_N, 1), lambda i: (i, 0)),
        out_shape=jax.ShapeDtypeStruct((NP, 1), jnp.float32),
    )(degP)


def _node_update(aggP, inv_deg, h, Wroot, WihT, WhhT, bconv_r, bih_r, bhh_r):
    def body(p_ref, iv_ref, h_ref, wr_ref, wi_ref, wh_ref, bc_ref, bi_ref,
             bh_ref, o_ref):
        i = pl.program_id(0)
        h_ = h_ref[...]
        agg = (p_ref[0] + p_ref[1]) * iv_ref[...]
        m = _leaky(
            agg
            + jnp.dot(h_, wr_ref[...], preferred_element_type=jnp.float32)
            + bc_ref[...]
        )
        gi = jnp.dot(m, wi_ref[...], preferred_element_type=jnp.float32) + bi_ref[...]
        gh = jnp.dot(h_, wh_ref[...], preferred_element_type=jnp.float32) + bh_ref[...]
        r = jax.nn.sigmoid(gi[:, 0:DIM] + gh[:, 0:DIM])
        z = jax.nn.sigmoid(gi[:, DIM:2 * DIM] + gh[:, DIM:2 * DIM])
        n = jnp.tanh(gi[:, 2 * DIM:3 * DIM] + r * gh[:, 2 * DIM:3 * DIM])
        hn = (1.0 - z) * n + z * h_
        rows = i * BLK_N + lax.broadcasted_iota(jnp.int32, (BLK_N, 1), 0)
        o_ref[...] = jnp.where(rows < N, hn, 0.0)

    g3 = 3 * DIM
    return pl.pallas_call(
        body,
        grid=(NP // BLK_N,),
        in_specs=[
            pl.BlockSpec((NC, BLK_N, DIM), lambda i: (0, i, 0)),
            pl.BlockSpec((BLK_N, 1), lambda i: (i, 0)),
            pl.BlockSpec((BLK_N, DIM), lambda i: (i, 0)),
            pl.BlockSpec((DIM, DIM), lambda i: (0, 0)),
            pl.BlockSpec((DIM, g3), lambda i: (0, 0)),
            pl.BlockSpec((DIM, g3), lambda i: (0, 0)),
            pl.BlockSpec((1, DIM), lambda i: (0, 0)),
            pl.BlockSpec((1, g3), lambda i: (0, 0)),
            pl.BlockSpec((1, g3), lambda i: (0, 0)),
        ],
        out_specs=pl.BlockSpec((BLK_N, DIM), lambda i: (i, 0)),
        out_shape=jax.ShapeDtypeStruct((NP, DIM), jnp.float32),
    )(aggP, inv_deg, h, Wroot, WihT, WhhT, bconv_r, bih_r, bhh_r)


def _set2set_out(h, batch_col, bihs_r, bhhs_r, WqT, WrT, bout_r):
    def body(h_ref, b_ref, bi_ref, bh_ref, wq_ref, wr_ref, bo_ref, o_ref):
        h_ = h_ref[...]
        bb = b_ref[...]
        g = bi_ref[...] + bh_ref[...]
        ig = jax.nn.sigmoid(g[:, 0:DIM])
        fg = jax.nn.sigmoid(g[:, DIM:2 * DIM])
        gg = jnp.tanh(g[:, 2 * DIM:3 * DIM])
        og = jax.nn.sigmoid(g[:, 3 * DIM:4 * DIM])
        cs = ig * gg + fg * 0.0
        q_row = og * jnp.tanh(cs)
        e = jnp.sum(h_ * q_row, axis=1, keepdims=True)
        ids = lax.broadcasted_iota(jnp.int32, (NP, B), 1)
        oh = (bb == ids).astype(jnp.float32)
        neg = jnp.float32(-1e30)
        emax_b = jnp.max(jnp.where(oh > 0, e, neg), axis=0, keepdims=True)
        emax_b = jnp.where(emax_b > neg * 0.5, emax_b, 0.0)
        emax_n = jnp.sum(oh * emax_b, axis=1, keepdims=True)
        valid = jnp.sum(oh, axis=1, keepdims=True)
        a_un = jnp.exp(e - emax_n) * valid
        denom_b = lax.dot_general(oh, a_un, (((0,), (0,)), ((), ())),
                                  preferred_element_type=jnp.float32)
        denom_n = jnp.dot(oh, denom_b, preferred_element_type=jnp.float32)
        a = a_un / jnp.where(denom_n > 0, denom_n, 1.0)
        r_vec = lax.dot_general(oh, a * h_, (((0,), (0,)), ((), ())),
                                preferred_element_type=jnp.float32)
        o_ref[...] = (
            jnp.dot(q_row, wq_ref[...], preferred_element_type=jnp.float32)
            + jnp.dot(r_vec, wr_ref[...], preferred_element_type=jnp.float32)
            + bo_ref[...]
        )

    return pl.pallas_call(
        body,
        out_shape=jax.ShapeDtypeStruct((B, OUT_DIM), jnp.float32),
    )(h, batch_col, bihs_r, bhhs_r, WqT, WrT, bout_r)


_R_EXPAND = np.repeat(np.eye(DIM, dtype=np.float32), DIM, axis=1)
_S_FOLD = (np.arange(128)[:, None] % DIM == np.arange(DIM)[None, :]
           ).astype(np.float32)


def kernel(x, edge_index, edge_attr, batch, W0, b0, Wn1, bn1, Wn2, bn2,
           Wroot, bconv, Wih, Whh, bih, bhh, Wih_s, Whh_s, bih_s, bhh_s,
           Wout, bout):
    src = edge_index[0]
    dst = edge_index[1]
    pad_idx = jnp.full((EP - E,), N, jnp.int32)
    srcp = jnp.concatenate([src, pad_idx])
    dst3 = jnp.concatenate([dst, pad_idx]).reshape(NW, NCHUNK, CHUNK)

    xp = jnp.pad(x, ((0, NP - N), (0, 0)))
    eap = jnp.pad(edge_attr, ((0, EP - E), (0, 0)))
    batch_col = jnp.pad(batch, (0, NP - N), constant_values=B).reshape(NP, 1)

    Wn2T = Wn2.T
    Bn2 = bn2.reshape(DIM, DIM)
    Rm = jnp.asarray(_R_EXPAND)
    Sm = jnp.asarray(_S_FOLD)
    zrow = jnp.zeros((SP, DIM), jnp.float32)
    onesE = jnp.ones((EP, DIM), jnp.float32)

    h0 = _row_matmul_act(xp, W0.T, b0.reshape(1, DIM), N, BLK_N, True)
    hid = _row_matmul_act(eap, Wn1.T, bn1.reshape(1, DIM), EP, BLK_E, True)

    degP = _sc_scatter_add(onesE, dst3, zrow)
    inv_deg = _deg_finalize(degP)

    WihT = Wih.T
    WhhT = Whh.T
    bconv_r = bconv.reshape(1, DIM)
    bih_r = bih.reshape(1, 3 * DIM)
    bhh_r = bhh.reshape(1, 3 * DIM)

    def step(_, h):
        xj = _sc_gather(h, srcp)
        msg = _msg_bilinear(xj, hid, Rm, Wn2T, Sm, Bn2)
        aggP = _sc_scatter_add(msg, dst3, zrow)
        return _node_update(aggP, inv_deg, h, Wroot, WihT, WhhT,
                            bconv_r, bih_r, bhh_r)

    h = lax.fori_loop(0, 6, step, h0)

    WoutT = Wout.T
    return _set2set_out(h, batch_col,
                        bih_s.reshape(1, 4 * DIM), bhh_s.reshape(1, 4 * DIM),
                        WoutT[:DIM], WoutT[DIM:], bout.reshape(1, OUT_DIM))


# speedup vs baseline: 2.4550x; 1.0978x over previous
import functools

import jax
import jax.numpy as jnp
import numpy as np
from jax import lax
from jax.experimental import pallas as pl
from jax.experimental.pallas import tpu as pltpu
from jax.experimental.pallas import tpu_sc as plsc

N = 10000
E = 160000
F_IN = 14
DIM = 32
B = 64
OUT_DIM = 16

NC = 2
NS = 16
NW = NC * NS
NP = 10240
EP = 163840
EPW = EP // NW
CHUNK = 128
NCHUNK = EPW // CHUNK
SP = NP // NS

BLK_E = 1024
BLK_N = 1024

_SC_MESH = dict(core_axis_name="c", subcore_axis_name="s")
_SC_PARAMS = pltpu.CompilerParams(use_tc_tiling_on_sc=False)


def _leaky(v):
    return jnp.where(v >= 0, v, 0.01 * v)


_GNB = 4


def _sc_gather(table, idx, half_off, n_edges):
    epw = n_edges // NW
    nchunk = epw // CHUNK

    @functools.partial(
        pl.kernel,
        out_type=jax.ShapeDtypeStruct((n_edges, DIM), jnp.float32),
        mesh=plsc.VectorSubcoreMesh(**_SC_MESH),
        compiler_params=_SC_PARAMS,
        scratch_types=[
            pltpu.VMEM((epw,), jnp.int32),
            pltpu.VMEM((_GNB, CHUNK, DIM), jnp.float32),
            pltpu.SemaphoreType.DMA((_GNB,)),
            pltpu.SemaphoreType.DMA((_GNB,)),
        ],
    )
    def k(table_hbm, idx_hbm, out_hbm, idx_v, rows_v, gsem, osem):
        wid = lax.axis_index("s") * NC + lax.axis_index("c")
        base = wid * epw
        pltpu.sync_copy(idx_hbm.at[pl.ds(half_off + base, epw)], idx_v)

        def start(j):
            return pltpu.async_copy(
                table_hbm.at[idx_v.at[pl.ds(j * CHUNK, CHUNK)]],
                rows_v.at[j % _GNB], gsem.at[j % _GNB],
            )

        gd = [None] * _GNB
        od = [None] * _GNB
        for j in range(min(_GNB - 1, nchunk)):
            gd[j % _GNB] = start(j)
        for j in range(nchunk):
            b = j % _GNB
            gd[b].wait()
            gd[b] = None
            od[b] = pltpu.async_copy(
                rows_v.at[b], out_hbm.at[pl.ds(base + j * CHUNK, CHUNK)],
                osem.at[b],
            )
            nj = j + _GNB - 1
            if nj < nchunk:
                nb = nj % _GNB
                if od[nb] is not None:
                    od[nb].wait()
                    od[nb] = None
                gd[nb] = start(nj)
        for b in range(_GNB):
            if od[b] is not None:
                od[b].wait()

    return k(table, idx)


_SNB = 4


def _sc_scatter_add(msg, dst3, zrow):
    n_edges = msg.shape[0]
    epw = n_edges // NW
    nchunk = epw // CHUNK

    @functools.partial(
        pl.kernel,
        out_type=jax.ShapeDtypeStruct((NC, NP, DIM), jnp.float32),
        mesh=plsc.VectorSubcoreMesh(**_SC_MESH),
        compiler_params=_SC_PARAMS,
        scratch_types=[
            pltpu.VMEM((nchunk, CHUNK), jnp.int32),
            pltpu.VMEM((_SNB, CHUNK, DIM), jnp.float32),
            pltpu.VMEM_SHARED((NP, DIM), jnp.float32),
            pltpu.SemaphoreType.DMA((_SNB,)),
            pltpu.SemaphoreType.DMA((_SNB,)),
        ],
    )
    def k(msg_hbm, dst_hbm, z_hbm, out_hbm, idx_v, mbuf, acc, msem, asem):
        cid = lax.axis_index("c")
        sid = lax.axis_index("s")
        wid = sid * NC + cid
        pltpu.sync_copy(z_hbm, acc.at[pl.ds(sid * SP, SP)])
        pltpu.sync_copy(dst_hbm.at[wid], idx_v)
        plsc.subcore_barrier()
        base = wid * epw

        def load(j):
            return pltpu.async_copy(
                msg_hbm.at[pl.ds(base + j * CHUNK, CHUNK)],
                mbuf.at[j % _SNB], msem.at[j % _SNB],
            )

        md = [None] * _SNB
        ad = [None] * _SNB
        for j in range(min(_SNB - 1, nchunk)):
            md[j % _SNB] = load(j)
        for j in range(nchunk):
            b = j % _SNB
            md[b].wait()
            md[b] = None
            ad[b] = pltpu.async_copy(
                mbuf.at[b], acc.at[idx_v.at[j]], asem.at[b], add=True
            )
            nj = j + _SNB - 1
            if nj < nchunk:
                nb = nj % _SNB
                if ad[nb] is not None:
                    ad[nb].wait()
                    ad[nb] = None
                md[nb] = load(nj)
        for b in range(_SNB):
            if ad[b] is not None:
                ad[b].wait()
        plsc.subcore_barrier()
        pltpu.sync_copy(
            acc.at[pl.ds(sid * SP, SP)], out_hbm.at[cid, pl.ds(sid * SP, SP)]
        )

    return k(msg, dst3, zrow)


def _row_matmul_act(xp, WT, brow, n_valid, blk, act):
    rows_total, f_in = xp.shape
    f_out = WT.shape[1]

    def body(x_ref, w_ref, b_ref, o_ref):
        i = pl.program_id(0)
        v = jnp.dot(x_ref[...], w_ref[...], preferred_element_type=jnp.float32)
        v = v + b_ref[...]
        if act:
            v = _leaky(v)
        rows = i * blk + lax.broadcasted_iota(jnp.int32, (blk, 1), 0)
        o_ref[...] = jnp.where(rows < n_valid, v, 0.0)

    return pl.pallas_call(
        body,
        grid=(rows_total // blk,),
        in_specs=[
            pl.BlockSpec((blk, f_in), lambda i: (i, 0)),
            pl.BlockSpec((f_in, f_out), lambda i: (0, 0)),
            pl.BlockSpec((1, f_out), lambda i: (0, 0)),
        ],
        out_specs=pl.BlockSpec((blk, f_out), lambda i: (i, 0)),
        out_shape=jax.ShapeDtypeStruct((rows_total, f_out), jnp.float32),
    )(xp, WT, brow)


def _msg_bilinear(xj, hid, Rm, Wn2T, Sm, Bn2, hoff=0):
    def body(x_ref, h_ref, r_ref, w_ref, s_ref, b_ref, o_ref):
        xb = x_ref[...]
        y = jnp.dot(h_ref[...], w_ref[...], preferred_element_type=jnp.float32)
        z = jnp.dot(xb, r_ref[...], preferred_element_type=jnp.float32) * y
        for wdt in (512, 256, 128):
            z = z[:, :wdt] + z[:, wdt:2 * wdt]
        o_ref[...] = (
            jnp.dot(z, s_ref[...], preferred_element_type=jnp.float32)
            + jnp.dot(xb, b_ref[...], preferred_element_type=jnp.float32)
        )

    d2 = DIM * DIM
    n_edges = xj.shape[0]
    return pl.pallas_call(
        body,
        grid=(n_edges // BLK_E,),
        in_specs=[
            pl.BlockSpec((BLK_E, DIM), lambda i: (i, 0)),
            pl.BlockSpec((BLK_E, DIM), lambda i: (i + hoff, 0)),
            pl.BlockSpec((DIM, d2), lambda i: (0, 0)),
            pl.BlockSpec((DIM, d2), lambda i: (0, 0)),
            pl.BlockSpec((128, DIM), lambda i: (0, 0)),
            pl.BlockSpec((DIM, DIM), lambda i: (0, 0)),
        ],
        out_specs=pl.BlockSpec((BLK_E, DIM), lambda i: (i, 0)),
        out_shape=jax.ShapeDtypeStruct((n_edges, DIM), jnp.float32),
    )(xj, hid, Rm, Wn2T, Sm, Bn2)


def _deg_finalize(degP):
    def body(p_ref, o_ref):
        deg = p_ref[0, :, 0:1] + p_ref[1, :, 0:1]
        o_ref[...] = 1.0 / jnp.maximum(deg, 1.0)

    return pl.pallas_call(
        body,
        grid=(NP // BLK_N,),
        in_specs=[pl.BlockSpec((NC, BLK_N, DIM), lambda i: (0, i, 0))],
        out_specs=pl.BlockSpec((BLK_N, 1), lambda i: (i, 0)),
        out_shape=jax.ShapeDtypeStruct((NP, 1), jnp.float32),
    )(degP)


def _node_update(aggP, aggQ, inv_deg, h, Wroot, WihT, WhhT, bconv_r, bih_r,
                 bhh_r):
    def body(p_ref, q_ref, iv_ref, h_ref, wr_ref, wi_ref, wh_ref, bc_ref,
             bi_ref, bh_ref, o_ref):
        i = pl.program_id(0)
        h_ = h_ref[...]
        agg = (p_ref[0] + p_ref[1] + q_ref[0] + q_ref[1]) * iv_ref[...]
        m = _leaky(
            agg
            + jnp.dot(h_, wr_ref[...], preferred_element_type=jnp.float32)
            + bc_ref[...]
        )
        gi = jnp.dot(m, wi_ref[...], preferred_element_type=jnp.float32) + bi_ref[...]
        gh = jnp.dot(h_, wh_ref[...], preferred_element_type=jnp.float32) + bh_ref[...]
        r = jax.nn.sigmoid(gi[:, 0:DIM] + gh[:, 0:DIM])
        z = jax.nn.sigmoid(gi[:, DIM:2 * DIM] + gh[:, DIM:2 * DIM])
        n = jnp.tanh(gi[:, 2 * DIM:3 * DIM] + r * gh[:, 2 * DIM:3 * DIM])
        hn = (1.0 - z) * n + z * h_
        rows = i * BLK_N + lax.broadcasted_iota(jnp.int32, (BLK_N, 1), 0)
        o_ref[...] = jnp.where(rows < N, hn, 0.0)

    g3 = 3 * DIM
    return pl.pallas_call(
        body,
        grid=(NP // BLK_N,),
        in_specs=[
            pl.BlockSpec((NC, BLK_N, DIM), lambda i: (0, i, 0)),
            pl.BlockSpec((NC, BLK_N, DIM), lambda i: (0, i, 0)),
            pl.BlockSpec((BLK_N, 1), lambda i: (i, 0)),
            pl.BlockSpec((BLK_N, DIM), lambda i: (i, 0)),
            pl.BlockSpec((DIM, DIM), lambda i: (0, 0)),
            pl.BlockSpec((DIM, g3), lambda i: (0, 0)),
            pl.BlockSpec((DIM, g3), lambda i: (0, 0)),
            pl.BlockSpec((1, DIM), lambda i: (0, 0)),
            pl.BlockSpec((1, g3), lambda i: (0, 0)),
            pl.BlockSpec((1, g3), lambda i: (0, 0)),
        ],
        out_specs=pl.BlockSpec((BLK_N, DIM), lambda i: (i, 0)),
        out_shape=jax.ShapeDtypeStruct((NP, DIM), jnp.float32),
    )(aggP, aggQ, inv_deg, h, Wroot, WihT, WhhT, bconv_r, bih_r, bhh_r)


def _set2set_out(h, batch_col, bihs_r, bhhs_r, WqT, WrT, bout_r):
    def body(h_ref, b_ref, bi_ref, bh_ref, wq_ref, wr_ref, bo_ref, o_ref):
        h_ = h_ref[...]
        bb = b_ref[...]
        g = bi_ref[...] + bh_ref[...]
        ig = jax.nn.sigmoid(g[:, 0:DIM])
        fg = jax.nn.sigmoid(g[:, DIM:2 * DIM])
        gg = jnp.tanh(g[:, 2 * DIM:3 * DIM])
        og = jax.nn.sigmoid(g[:, 3 * DIM:4 * DIM])
        cs = ig * gg + fg * 0.0
        q_row = og * jnp.tanh(cs)
        e = jnp.sum(h_ * q_row, axis=1, keepdims=True)
        ids = lax.broadcasted_iota(jnp.int32, (NP, B), 1)
        oh = (bb == ids).astype(jnp.float32)
        neg = jnp.float32(-1e30)
        emax_b = jnp.max(jnp.where(oh > 0, e, neg), axis=0, keepdims=True)
        emax_b = jnp.where(emax_b > neg * 0.5, emax_b, 0.0)
        emax_n = jnp.sum(oh * emax_b, axis=1, keepdims=True)
        valid = jnp.sum(oh, axis=1, keepdims=True)
        a_un = jnp.exp(e - emax_n) * valid
        denom_b = lax.dot_general(oh, a_un, (((0,), (0,)), ((), ())),
                                  preferred_element_type=jnp.float32)
        denom_n = jnp.dot(oh, denom_b, preferred_element_type=jnp.float32)
        a = a_un / jnp.where(denom_n > 0, denom_n, 1.0)
        r_vec = lax.dot_general(oh, a * h_, (((0,), (0,)), ((), ())),
                                preferred_element_type=jnp.float32)
        o_ref[...] = (
            jnp.dot(q_row, wq_ref[...], preferred_element_type=jnp.float32)
            + jnp.dot(r_vec, wr_ref[...], preferred_element_type=jnp.float32)
            + bo_ref[...]
        )

    return pl.pallas_call(
        body,
        out_shape=jax.ShapeDtypeStruct((B, OUT_DIM), jnp.float32),
    )(h, batch_col, bihs_r, bhhs_r, WqT, WrT, bout_r)


_R_EXPAND = np.repeat(np.eye(DIM, dtype=np.float32), DIM, axis=1)
_S_FOLD = (np.arange(128)[:, None] % DIM == np.arange(DIM)[None, :]
           ).astype(np.float32)


def kernel(x, edge_index, edge_attr, batch, W0, b0, Wn1, bn1, Wn2, bn2,
           Wroot, bconv, Wih, Whh, bih, bhh, Wih_s, Whh_s, bih_s, bhh_s,
           Wout, bout):
    src = edge_index[0]
    dst = edge_index[1]
    pad_idx = jnp.full((EP - E,), N, jnp.int32)
    srcp = jnp.concatenate([src, pad_idx])
    dstp = jnp.concatenate([dst, pad_idx])
    dst3 = dstp.reshape(NW, NCHUNK, CHUNK)
    ep2 = EP // 2
    dst3A = dstp[:ep2].reshape(NW, NCHUNK // 2, CHUNK)
    dst3B = dstp[ep2:].reshape(NW, NCHUNK // 2, CHUNK)

    xp = jnp.pad(x, ((0, NP - N), (0, 0)))
    eap = jnp.pad(edge_attr, ((0, EP - E), (0, 0)))
    batch_col = jnp.pad(batch, (0, NP - N), constant_values=B).reshape(NP, 1)

    Wn2T = Wn2.T
    Bn2 = bn2.reshape(DIM, DIM)
    Rm = jnp.asarray(_R_EXPAND)
    Sm = jnp.asarray(_S_FOLD)
    zrow = jnp.zeros((SP, DIM), jnp.float32)
    onesE = jnp.ones((EP, DIM), jnp.float32)

    h0 = _row_matmul_act(xp, W0.T, b0.reshape(1, DIM), N, BLK_N, True)
    hid = _row_matmul_act(eap, Wn1.T, bn1.reshape(1, DIM), EP, BLK_E, True)

    degP = _sc_scatter_add(onesE, dst3, zrow)
    inv_deg = _deg_finalize(degP)

    WihT = Wih.T
    WhhT = Whh.T
    bconv_r = bconv.reshape(1, DIM)
    bih_r = bih.reshape(1, 3 * DIM)
    bhh_r = bhh.reshape(1, 3 * DIM)

    def step(_, h):
        xjA = _sc_gather(h, srcp, 0, ep2)
        xjB = _sc_gather(h, srcp, ep2, ep2)
        msgA = _msg_bilinear(xjA, hid, Rm, Wn2T, Sm, Bn2, 0)
        aggA = _sc_scatter_add(msgA, dst3A, zrow)
        msgB = _msg_bilinear(xjB, hid, Rm, Wn2T, Sm, Bn2, ep2 // BLK_E)
        aggB = _sc_scatter_add(msgB, dst3B, zrow)
        return _node_update(aggA, aggB, inv_deg, h, Wroot, WihT, WhhT,
                            bconv_r, bih_r, bhh_r)

    h = lax.fori_loop(0, 6, step, h0)

    WoutT = Wout.T
    return _set2set_out(h, batch_col,
                        bih_s.reshape(1, 4 * DIM), bhh_s.reshape(1, 4 * DIM),
                        WoutT[:DIM], WoutT[DIM:], bout.reshape(1, OUT_DIM))


# speedup vs baseline: 2.7786x; 1.1318x over previous
import functools

import jax
import jax.numpy as jnp
import numpy as np
from jax import lax
from jax.experimental import pallas as pl
from jax.experimental.pallas import tpu as pltpu
from jax.experimental.pallas import tpu_sc as plsc

N = 10000
E = 160000
F_IN = 14
DIM = 32
B = 64
OUT_DIM = 16

NC = 2
NS = 16
NW = NC * NS
NP = 10240
EP = 163840
EPW = EP // NW
CHUNK = 128
NCHUNK = EPW // CHUNK
SP = NP // NS

BLK_E = 2048
BLK_N = 1024

_SC_MESH = dict(core_axis_name="c", subcore_axis_name="s")
_SC_PARAMS = pltpu.CompilerParams(use_tc_tiling_on_sc=False)


def _leaky(v):
    return jnp.where(v >= 0, v, 0.01 * v)


_GNB = 4


def _sc_gather(table, idx, half_off, n_edges):
    epw = n_edges // NW
    nchunk = epw // CHUNK

    @functools.partial(
        pl.kernel,
        out_type=jax.ShapeDtypeStruct((n_edges, DIM), jnp.float32),
        mesh=plsc.VectorSubcoreMesh(**_SC_MESH),
        compiler_params=_SC_PARAMS,
        scratch_types=[
            pltpu.VMEM((epw,), jnp.int32),
            pltpu.VMEM((_GNB, CHUNK, DIM), jnp.float32),
            pltpu.VMEM_SHARED((NP, DIM), jnp.float32),
            pltpu.SemaphoreType.DMA((_GNB,)),
            pltpu.SemaphoreType.DMA((_GNB,)),
        ],
    )
    def k(table_hbm, idx_hbm, out_hbm, idx_v, rows_v, stable, gsem, osem):
        sid = lax.axis_index("s")
        wid = sid * NC + lax.axis_index("c")
        base = wid * epw
        pltpu.sync_copy(table_hbm.at[pl.ds(sid * SP, SP)],
                        stable.at[pl.ds(sid * SP, SP)])
        pltpu.sync_copy(idx_hbm.at[pl.ds(half_off + base, epw)], idx_v)
        plsc.subcore_barrier()

        def start(j):
            return pltpu.async_copy(
                stable.at[idx_v.at[pl.ds(j * CHUNK, CHUNK)]],
                rows_v.at[j % _GNB], gsem.at[j % _GNB],
            )

        gd = [None] * _GNB
        od = [None] * _GNB
        for j in range(min(_GNB - 1, nchunk)):
            gd[j % _GNB] = start(j)
        for j in range(nchunk):
            b = j % _GNB
            gd[b].wait()
            gd[b] = None
            od[b] = pltpu.async_copy(
                rows_v.at[b], out_hbm.at[pl.ds(base + j * CHUNK, CHUNK)],
                osem.at[b],
            )
            nj = j + _GNB - 1
            if nj < nchunk:
                nb = nj % _GNB
                if od[nb] is not None:
                    od[nb].wait()
                    od[nb] = None
                gd[nb] = start(nj)
        for b in range(_GNB):
            if od[b] is not None:
                od[b].wait()

    return k(table, idx)


_SNB = 4


def _sc_scatter_add(msg, dst3, zrow):
    n_edges = msg.shape[0]
    epw = n_edges // NW
    nchunk = epw // CHUNK

    @functools.partial(
        pl.kernel,
        out_type=jax.ShapeDtypeStruct((NC, NP, DIM), jnp.float32),
        mesh=plsc.VectorSubcoreMesh(**_SC_MESH),
        compiler_params=_SC_PARAMS,
        scratch_types=[
            pltpu.VMEM((nchunk, CHUNK), jnp.int32),
            pltpu.VMEM((_SNB, CHUNK, DIM), jnp.float32),
            pltpu.VMEM_SHARED((NP, DIM), jnp.float32),
            pltpu.SemaphoreType.DMA((_SNB,)),
            pltpu.SemaphoreType.DMA((_SNB,)),
        ],
    )
    def k(msg_hbm, dst_hbm, z_hbm, out_hbm, idx_v, mbuf, acc, msem, asem):
        cid = lax.axis_index("c")
        sid = lax.axis_index("s")
        wid = sid * NC + cid
        pltpu.sync_copy(z_hbm, acc.at[pl.ds(sid * SP, SP)])
        pltpu.sync_copy(dst_hbm.at[wid], idx_v)
        plsc.subcore_barrier()
        base = wid * epw

        def load(j):
            return pltpu.async_copy(
                msg_hbm.at[pl.ds(base + j * CHUNK, CHUNK)],
                mbuf.at[j % _SNB], msem.at[j % _SNB],
            )

        md = [None] * _SNB
        ad = [None] * _SNB
        for j in range(min(_SNB - 1, nchunk)):
            md[j % _SNB] = load(j)
        for j in range(nchunk):
            b = j % _SNB
            md[b].wait()
            md[b] = None
            ad[b] = pltpu.async_copy(
                mbuf.at[b], acc.at[idx_v.at[j]], asem.at[b], add=True
            )
            nj = j + _SNB - 1
            if nj < nchunk:
                nb = nj % _SNB
                if ad[nb] is not None:
                    ad[nb].wait()
                    ad[nb] = None
                md[nb] = load(nj)
        for b in range(_SNB):
            if ad[b] is not None:
                ad[b].wait()
        plsc.subcore_barrier()
        pltpu.sync_copy(
            acc.at[pl.ds(sid * SP, SP)], out_hbm.at[cid, pl.ds(sid * SP, SP)]
        )

    return k(msg, dst3, zrow)


def _row_matmul_act(xp, WT, brow, n_valid, blk, act):
    rows_total, f_in = xp.shape
    f_out = WT.shape[1]

    def body(x_ref, w_ref, b_ref, o_ref):
        i = pl.program_id(0)
        v = jnp.dot(x_ref[...], w_ref[...], preferred_element_type=jnp.float32)
        v = v + b_ref[...]
        if act:
            v = _leaky(v)
        rows = i * blk + lax.broadcasted_iota(jnp.int32, (blk, 1), 0)
        o_ref[...] = jnp.where(rows < n_valid, v, 0.0)

    return pl.pallas_call(
        body,
        grid=(rows_total // blk,),
        in_specs=[
            pl.BlockSpec((blk, f_in), lambda i: (i, 0)),
            pl.BlockSpec((f_in, f_out), lambda i: (0, 0)),
            pl.BlockSpec((1, f_out), lambda i: (0, 0)),
        ],
        out_specs=pl.BlockSpec((blk, f_out), lambda i: (i, 0)),
        out_shape=jax.ShapeDtypeStruct((rows_total, f_out), jnp.float32),
    )(xp, WT, brow)


def _msg_bilinear(xj, hid, Rm, Wn2T, Sm, Bn2, hoff=0):
    def body(x_ref, h_ref, r_ref, w_ref, s_ref, b_ref, o_ref):
        xb = x_ref[...]
        y = jnp.dot(h_ref[...], w_ref[...], preferred_element_type=jnp.float32)
        z = jnp.dot(xb, r_ref[...], preferred_element_type=jnp.float32) * y
        for wdt in (512, 256, 128):
            z = z[:, :wdt] + z[:, wdt:2 * wdt]
        o_ref[...] = (
            jnp.dot(z, s_ref[...], preferred_element_type=jnp.float32)
            + jnp.dot(xb, b_ref[...], preferred_element_type=jnp.float32)
        )

    d2 = DIM * DIM
    n_edges = xj.shape[0]
    return pl.pallas_call(
        body,
        grid=(n_edges // BLK_E,),
        in_specs=[
            pl.BlockSpec((BLK_E, DIM), lambda i: (i, 0)),
            pl.BlockSpec((BLK_E, DIM), lambda i: (i + hoff, 0)),
            pl.BlockSpec((DIM, d2), lambda i: (0, 0)),
            pl.BlockSpec((DIM, d2), lambda i: (0, 0)),
            pl.BlockSpec((128, DIM), lambda i: (0, 0)),
            pl.BlockSpec((DIM, DIM), lambda i: (0, 0)),
        ],
        out_specs=pl.BlockSpec((BLK_E, DIM), lambda i: (i, 0)),
        out_shape=jax.ShapeDtypeStruct((n_edges, DIM), jnp.float32),
    )(xj, hid, Rm, Wn2T, Sm, Bn2)


def _deg_finalize(degP):
    def body(p_ref, o_ref):
        deg = p_ref[0, :, 0:1] + p_ref[1, :, 0:1]
        o_ref[...] = 1.0 / jnp.maximum(deg, 1.0)

    return pl.pallas_call(
        body,
        grid=(NP // BLK_N,),
        in_specs=[pl.BlockSpec((NC, BLK_N, DIM), lambda i: (0, i, 0))],
        out_specs=pl.BlockSpec((BLK_N, 1), lambda i: (i, 0)),
        out_shape=jax.ShapeDtypeStruct((NP, 1), jnp.float32),
    )(degP)


def _node_update(aggP, aggQ, inv_deg, h, Wroot, WihT, WhhT, bconv_r, bih_r,
                 bhh_r):
    def body(p_ref, q_ref, iv_ref, h_ref, wr_ref, wi_ref, wh_ref, bc_ref,
             bi_ref, bh_ref, o_ref):
        i = pl.program_id(0)
        h_ = h_ref[...]
        agg = (p_ref[0] + p_ref[1] + q_ref[0] + q_ref[1]) * iv_ref[...]
        m = _leaky(
            agg
            + jnp.dot(h_, wr_ref[...], preferred_element_type=jnp.float32)
            + bc_ref[...]
        )
        gi = jnp.dot(m, wi_ref[...], preferred_element_type=jnp.float32) + bi_ref[...]
        gh = jnp.dot(h_, wh_ref[...], preferred_element_type=jnp.float32) + bh_ref[...]
        r = jax.nn.sigmoid(gi[:, 0:DIM] + gh[:, 0:DIM])
        z = jax.nn.sigmoid(gi[:, DIM:2 * DIM] + gh[:, DIM:2 * DIM])
        n = jnp.tanh(gi[:, 2 * DIM:3 * DIM] + r * gh[:, 2 * DIM:3 * DIM])
        hn = (1.0 - z) * n + z * h_
        rows = i * BLK_N + lax.broadcasted_iota(jnp.int32, (BLK_N, 1), 0)
        o_ref[...] = jnp.where(rows < N, hn, 0.0)

    g3 = 3 * DIM
    return pl.pallas_call(
        body,
        grid=(NP // BLK_N,),
        in_specs=[
            pl.BlockSpec((NC, BLK_N, DIM), lambda i: (0, i, 0)),
            pl.BlockSpec((NC, BLK_N, DIM), lambda i: (0, i, 0)),
            pl.BlockSpec((BLK_N, 1), lambda i: (i, 0)),
            pl.BlockSpec((BLK_N, DIM), lambda i: (i, 0)),
            pl.BlockSpec((DIM, DIM), lambda i: (0, 0)),
            pl.BlockSpec((DIM, g3), lambda i: (0, 0)),
            pl.BlockSpec((DIM, g3), lambda i: (0, 0)),
            pl.BlockSpec((1, DIM), lambda i: (0, 0)),
            pl.BlockSpec((1, g3), lambda i: (0, 0)),
            pl.BlockSpec((1, g3), lambda i: (0, 0)),
        ],
        out_specs=pl.BlockSpec((BLK_N, DIM), lambda i: (i, 0)),
        out_shape=jax.ShapeDtypeStruct((NP, DIM), jnp.float32),
    )(aggP, aggQ, inv_deg, h, Wroot, WihT, WhhT, bconv_r, bih_r, bhh_r)


def _set2set_out(h, batch_col, bihs_r, bhhs_r, WqT, WrT, bout_r):
    def body(h_ref, b_ref, bi_ref, bh_ref, wq_ref, wr_ref, bo_ref, o_ref):
        h_ = h_ref[...]
        bb = b_ref[...]
        g = bi_ref[...] + bh_ref[...]
        ig = jax.nn.sigmoid(g[:, 0:DIM])
        fg = jax.nn.sigmoid(g[:, DIM:2 * DIM])
        gg = jnp.tanh(g[:, 2 * DIM:3 * DIM])
        og = jax.nn.sigmoid(g[:, 3 * DIM:4 * DIM])
        cs = ig * gg + fg * 0.0
        q_row = og * jnp.tanh(cs)
        e = jnp.sum(h_ * q_row, axis=1, keepdims=True)
        ids = lax.broadcasted_iota(jnp.int32, (NP, B), 1)
        oh = (bb == ids).astype(jnp.float32)
        neg = jnp.float32(-1e30)
        emax_b = jnp.max(jnp.where(oh > 0, e, neg), axis=0, keepdims=True)
        emax_b = jnp.where(emax_b > neg * 0.5, emax_b, 0.0)
        emax_n = jnp.sum(oh * emax_b, axis=1, keepdims=True)
        valid = jnp.sum(oh, axis=1, keepdims=True)
        a_un = jnp.exp(e - emax_n) * valid
        denom_b = lax.dot_general(oh, a_un, (((0,), (0,)), ((), ())),
                                  preferred_element_type=jnp.float32)
        denom_n = jnp.dot(oh, denom_b, preferred_element_type=jnp.float32)
        a = a_un / jnp.where(denom_n > 0, denom_n, 1.0)
        r_vec = lax.dot_general(oh, a * h_, (((0,), (0,)), ((), ())),
                                preferred_element_type=jnp.float32)
        o_ref[...] = (
            jnp.dot(q_row, wq_ref[...], preferred_element_type=jnp.float32)
            + jnp.dot(r_vec, wr_ref[...], preferred_element_type=jnp.float32)
            + bo_ref[...]
        )

    return pl.pallas_call(
        body,
        out_shape=jax.ShapeDtypeStruct((B, OUT_DIM), jnp.float32),
    )(h, batch_col, bihs_r, bhhs_r, WqT, WrT, bout_r)


_R_EXPAND = np.repeat(np.eye(DIM, dtype=np.float32), DIM, axis=1)
_S_FOLD = (np.arange(128)[:, None] % DIM == np.arange(DIM)[None, :]
           ).astype(np.float32)


def kernel(x, edge_index, edge_attr, batch, W0, b0, Wn1, bn1, Wn2, bn2,
           Wroot, bconv, Wih, Whh, bih, bhh, Wih_s, Whh_s, bih_s, bhh_s,
           Wout, bout):
    src = edge_index[0]
    dst = edge_index[1]
    pad_idx = jnp.full((EP - E,), N, jnp.int32)
    srcp = jnp.concatenate([src, pad_idx])
    dstp = jnp.concatenate([dst, pad_idx])
    dst3 = dstp.reshape(NW, NCHUNK, CHUNK)
    ep2 = EP // 2
    dst3A = dstp[:ep2].reshape(NW, NCHUNK // 2, CHUNK)
    dst3B = dstp[ep2:].reshape(NW, NCHUNK // 2, CHUNK)

    xp = jnp.pad(x, ((0, NP - N), (0, 0)))
    eap = jnp.pad(edge_attr, ((0, EP - E), (0, 0)))
    batch_col = jnp.pad(batch, (0, NP - N), constant_values=B).reshape(NP, 1)

    Wn2T = Wn2.T
    Bn2 = bn2.reshape(DIM, DIM)
    Rm = jnp.asarray(_R_EXPAND)
    Sm = jnp.asarray(_S_FOLD)
    zrow = jnp.zeros((SP, DIM), jnp.float32)
    onesE = jnp.ones((EP, DIM), jnp.float32)

    h0 = _row_matmul_act(xp, W0.T, b0.reshape(1, DIM), N, BLK_N, True)
    hid = _row_matmul_act(eap, Wn1.T, bn1.reshape(1, DIM), EP, BLK_E, True)

    degP = _sc_scatter_add(onesE, dst3, zrow)
    inv_deg = _deg_finalize(degP)

    WihT = Wih.T
    WhhT = Whh.T
    bconv_r = bconv.reshape(1, DIM)
    bih_r = bih.reshape(1, 3 * DIM)
    bhh_r = bhh.reshape(1, 3 * DIM)

    def step(_, h):
        xjA = _sc_gather(h, srcp, 0, ep2)
        xjB = _sc_gather(h, srcp, ep2, ep2)
        msgA = _msg_bilinear(xjA, hid, Rm, Wn2T, Sm, Bn2, 0)
        aggA = _sc_scatter_add(msgA, dst3A, zrow)
        msgB = _msg_bilinear(xjB, hid, Rm, Wn2T, Sm, Bn2, ep2 // BLK_E)
        aggB = _sc_scatter_add(msgB, dst3B, zrow)
        return _node_update(aggA, aggB, inv_deg, h, Wroot, WihT, WhhT,
                            bconv_r, bih_r, bhh_r)

    h = lax.fori_loop(0, 6, step, h0)

    WoutT = Wout.T
    return _set2set_out(h, batch_col,
                        bih_s.reshape(1, 4 * DIM), bhh_s.reshape(1, 4 * DIM),
                        WoutT[:DIM], WoutT[DIM:], bout.reshape(1, OUT_DIM))


# speedup vs baseline: 2.7851x; 1.0023x over previous
import functools

import jax
import jax.numpy as jnp
import numpy as np
from jax import lax
from jax.experimental import pallas as pl
from jax.experimental.pallas import tpu as pltpu
from jax.experimental.pallas import tpu_sc as plsc

N = 10000
E = 160000
F_IN = 14
DIM = 32
B = 64
OUT_DIM = 16

NC = 2
NS = 16
NW = NC * NS
NP = 10240
EP = 163840
EPW = EP // NW
CHUNK = 128
NCHUNK = EPW // CHUNK
SP = NP // NS

BLK_E = 2048
BLK_N = 1024

_SC_MESH = dict(core_axis_name="c", subcore_axis_name="s")
_SC_PARAMS = pltpu.CompilerParams(use_tc_tiling_on_sc=False)


def _leaky(v):
    return jnp.where(v >= 0, v, 0.01 * v)


_GNB = 4


def _sc_gather(table, idx, half_off, n_edges):
    epw = n_edges // NW
    nchunk = epw // CHUNK

    @functools.partial(
        pl.kernel,
        out_type=jax.ShapeDtypeStruct((n_edges, DIM), jnp.float32),
        mesh=plsc.VectorSubcoreMesh(**_SC_MESH),
        compiler_params=_SC_PARAMS,
        scratch_types=[
            pltpu.VMEM((epw,), jnp.int32),
            pltpu.VMEM((_GNB, CHUNK, DIM), jnp.float32),
            pltpu.VMEM_SHARED((NP, DIM), jnp.float32),
            pltpu.SemaphoreType.DMA((_GNB,)),
            pltpu.SemaphoreType.DMA((_GNB,)),
        ],
    )
    def k(table_hbm, idx_hbm, out_hbm, idx_v, rows_v, stable, gsem, osem):
        sid = lax.axis_index("s")
        wid = sid * NC + lax.axis_index("c")
        base = wid * epw
        pltpu.sync_copy(table_hbm.at[pl.ds(sid * SP, SP)],
                        stable.at[pl.ds(sid * SP, SP)])
        pltpu.sync_copy(idx_hbm.at[pl.ds(half_off + base, epw)], idx_v)
        plsc.subcore_barrier()

        def start(j):
            return pltpu.async_copy(
                stable.at[idx_v.at[pl.ds(j * CHUNK, CHUNK)]],
                rows_v.at[j % _GNB], gsem.at[j % _GNB],
            )

        gd = [None] * _GNB
        od = [None] * _GNB
        for j in range(min(_GNB - 1, nchunk)):
            gd[j % _GNB] = start(j)
        for j in range(nchunk):
            b = j % _GNB
            gd[b].wait()
            gd[b] = None
            od[b] = pltpu.async_copy(
                rows_v.at[b], out_hbm.at[pl.ds(base + j * CHUNK, CHUNK)],
                osem.at[b],
            )
            nj = j + _GNB - 1
            if nj < nchunk:
                nb = nj % _GNB
                if od[nb] is not None:
                    od[nb].wait()
                    od[nb] = None
                gd[nb] = start(nj)
        for b in range(_GNB):
            if od[b] is not None:
                od[b].wait()

    return k(table, idx)


_SNB = 4


def _sc_scatter_add(msg, dst3, zrow):
    n_edges = msg.shape[0]
    epw = n_edges // NW
    nchunk = epw // CHUNK

    @functools.partial(
        pl.kernel,
        out_type=jax.ShapeDtypeStruct((NC, NP, DIM), jnp.float32),
        mesh=plsc.VectorSubcoreMesh(**_SC_MESH),
        compiler_params=_SC_PARAMS,
        scratch_types=[
            pltpu.VMEM((nchunk, CHUNK), jnp.int32),
            pltpu.VMEM((_SNB, CHUNK, DIM), jnp.float32),
            pltpu.VMEM_SHARED((NP, DIM), jnp.float32),
            pltpu.SemaphoreType.DMA((_SNB,)),
            pltpu.SemaphoreType.DMA((_SNB,)),
        ],
    )
    def k(msg_hbm, dst_hbm, z_hbm, out_hbm, idx_v, mbuf, acc, msem, asem):
        cid = lax.axis_index("c")
        sid = lax.axis_index("s")
        wid = sid * NC + cid
        pltpu.sync_copy(z_hbm, acc.at[pl.ds(sid * SP, SP)])
        pltpu.sync_copy(dst_hbm.at[wid], idx_v)
        plsc.subcore_barrier()
        base = wid * epw

        def load(j):
            return pltpu.async_copy(
                msg_hbm.at[pl.ds(base + j * CHUNK, CHUNK)],
                mbuf.at[j % _SNB], msem.at[j % _SNB],
            )

        md = [None] * _SNB
        ad = [None] * _SNB
        for j in range(min(_SNB - 1, nchunk)):
            md[j % _SNB] = load(j)
        for j in range(nchunk):
            b = j % _SNB
            md[b].wait()
            md[b] = None
            ad[b] = pltpu.async_copy(
                mbuf.at[b], acc.at[idx_v.at[j]], asem.at[b], add=True
            )
            nj = j + _SNB - 1
            if nj < nchunk:
                nb = nj % _SNB
                if ad[nb] is not None:
                    ad[nb].wait()
                    ad[nb] = None
                md[nb] = load(nj)
        for b in range(_SNB):
            if ad[b] is not None:
                ad[b].wait()
        plsc.subcore_barrier()
        pltpu.sync_copy(
            acc.at[pl.ds(sid * SP, SP)], out_hbm.at[cid, pl.ds(sid * SP, SP)]
        )

    return k(msg, dst3, zrow)


def _row_matmul_act(xp, WT, brow, n_valid, blk, act):
    rows_total, f_in = xp.shape
    f_out = WT.shape[1]

    def body(x_ref, w_ref, b_ref, o_ref):
        i = pl.program_id(0)
        v = jnp.dot(x_ref[...], w_ref[...], preferred_element_type=jnp.float32)
        v = v + b_ref[...]
        if act:
            v = _leaky(v)
        rows = i * blk + lax.broadcasted_iota(jnp.int32, (blk, 1), 0)
        o_ref[...] = jnp.where(rows < n_valid, v, 0.0)

    return pl.pallas_call(
        body,
        grid=(rows_total // blk,),
        in_specs=[
            pl.BlockSpec((blk, f_in), lambda i: (i, 0)),
            pl.BlockSpec((f_in, f_out), lambda i: (0, 0)),
            pl.BlockSpec((1, f_out), lambda i: (0, 0)),
        ],
        out_specs=pl.BlockSpec((blk, f_out), lambda i: (i, 0)),
        out_shape=jax.ShapeDtypeStruct((rows_total, f_out), jnp.float32),
    )(xp, WT, brow)


def _msg_bilinear(xj, hid, Rm, Wn2T, Sm, Bn2, hoff=0):
    def body(x_ref, h_ref, r_ref, w_ref, s_ref, b_ref, o_ref):
        xb = x_ref[...]
        y = jnp.dot(h_ref[...], w_ref[...], preferred_element_type=jnp.float32)
        z = jnp.dot(xb, r_ref[...], preferred_element_type=jnp.float32) * y
        for wdt in (512, 256, 128):
            z = z[:, :wdt] + z[:, wdt:2 * wdt]
        o_ref[...] = (
            jnp.dot(z, s_ref[...], preferred_element_type=jnp.float32)
            + jnp.dot(xb, b_ref[...], preferred_element_type=jnp.float32)
        )

    d2 = DIM * DIM
    n_edges = EP // 2
    return pl.pallas_call(
        body,
        grid=(n_edges // BLK_E,),
        in_specs=[
            pl.BlockSpec((BLK_E, DIM), lambda i: (i + hoff, 0)),
            pl.BlockSpec((BLK_E, DIM), lambda i: (i + hoff, 0)),
            pl.BlockSpec((DIM, d2), lambda i: (0, 0)),
            pl.BlockSpec((DIM, d2), lambda i: (0, 0)),
            pl.BlockSpec((128, DIM), lambda i: (0, 0)),
            pl.BlockSpec((DIM, DIM), lambda i: (0, 0)),
        ],
        out_specs=pl.BlockSpec((BLK_E, DIM), lambda i: (i, 0)),
        out_shape=jax.ShapeDtypeStruct((n_edges, DIM), jnp.float32),
    )(xj, hid, Rm, Wn2T, Sm, Bn2)


def _deg_finalize(degP):
    def body(p_ref, o_ref):
        deg = p_ref[0, :, 0:1] + p_ref[1, :, 0:1]
        o_ref[...] = 1.0 / jnp.maximum(deg, 1.0)

    return pl.pallas_call(
        body,
        grid=(NP // BLK_N,),
        in_specs=[pl.BlockSpec((NC, BLK_N, DIM), lambda i: (0, i, 0))],
        out_specs=pl.BlockSpec((BLK_N, 1), lambda i: (i, 0)),
        out_shape=jax.ShapeDtypeStruct((NP, 1), jnp.float32),
    )(degP)


def _node_update(aggP, aggQ, inv_deg, h, Wroot, WihT, WhhT, bconv_r, bih_r,
                 bhh_r):
    def body(p_ref, q_ref, iv_ref, h_ref, wr_ref, wi_ref, wh_ref, bc_ref,
             bi_ref, bh_ref, o_ref):
        i = pl.program_id(0)
        h_ = h_ref[...]
        agg = (p_ref[0] + p_ref[1] + q_ref[0] + q_ref[1]) * iv_ref[...]
        m = _leaky(
            agg
            + jnp.dot(h_, wr_ref[...], preferred_element_type=jnp.float32)
            + bc_ref[...]
        )
        gi = jnp.dot(m, wi_ref[...], preferred_element_type=jnp.float32) + bi_ref[...]
        gh = jnp.dot(h_, wh_ref[...], preferred_element_type=jnp.float32) + bh_ref[...]
        r = jax.nn.sigmoid(gi[:, 0:DIM] + gh[:, 0:DIM])
        z = jax.nn.sigmoid(gi[:, DIM:2 * DIM] + gh[:, DIM:2 * DIM])
        n = jnp.tanh(gi[:, 2 * DIM:3 * DIM] + r * gh[:, 2 * DIM:3 * DIM])
        hn = (1.0 - z) * n + z * h_
        rows = i * BLK_N + lax.broadcasted_iota(jnp.int32, (BLK_N, 1), 0)
        o_ref[...] = jnp.where(rows < N, hn, 0.0)

    g3 = 3 * DIM
    return pl.pallas_call(
        body,
        grid=(NP // BLK_N,),
        in_specs=[
            pl.BlockSpec((NC, BLK_N, DIM), lambda i: (0, i, 0)),
            pl.BlockSpec((NC, BLK_N, DIM), lambda i: (0, i, 0)),
            pl.BlockSpec((BLK_N, 1), lambda i: (i, 0)),
            pl.BlockSpec((BLK_N, DIM), lambda i: (i, 0)),
            pl.BlockSpec((DIM, DIM), lambda i: (0, 0)),
            pl.BlockSpec((DIM, g3), lambda i: (0, 0)),
            pl.BlockSpec((DIM, g3), lambda i: (0, 0)),
            pl.BlockSpec((1, DIM), lambda i: (0, 0)),
            pl.BlockSpec((1, g3), lambda i: (0, 0)),
            pl.BlockSpec((1, g3), lambda i: (0, 0)),
        ],
        out_specs=pl.BlockSpec((BLK_N, DIM), lambda i: (i, 0)),
        out_shape=jax.ShapeDtypeStruct((NP, DIM), jnp.float32),
    )(aggP, aggQ, inv_deg, h, Wroot, WihT, WhhT, bconv_r, bih_r, bhh_r)


def _set2set_out(h, batch_col, bihs_r, bhhs_r, WqT, WrT, bout_r):
    def body(h_ref, b_ref, bi_ref, bh_ref, wq_ref, wr_ref, bo_ref, o_ref):
        h_ = h_ref[...]
        bb = b_ref[...]
        g = bi_ref[...] + bh_ref[...]
        ig = jax.nn.sigmoid(g[:, 0:DIM])
        fg = jax.nn.sigmoid(g[:, DIM:2 * DIM])
        gg = jnp.tanh(g[:, 2 * DIM:3 * DIM])
        og = jax.nn.sigmoid(g[:, 3 * DIM:4 * DIM])
        cs = ig * gg + fg * 0.0
        q_row = og * jnp.tanh(cs)
        e = jnp.sum(h_ * q_row, axis=1, keepdims=True)
        ids = lax.broadcasted_iota(jnp.int32, (NP, B), 1)
        oh = (bb == ids).astype(jnp.float32)
        neg = jnp.float32(-1e30)
        emax_b = jnp.max(jnp.where(oh > 0, e, neg), axis=0, keepdims=True)
        emax_b = jnp.where(emax_b > neg * 0.5, emax_b, 0.0)
        emax_n = jnp.sum(oh * emax_b, axis=1, keepdims=True)
        valid = jnp.sum(oh, axis=1, keepdims=True)
        a_un = jnp.exp(e - emax_n) * valid
        denom_b = lax.dot_general(oh, a_un, (((0,), (0,)), ((), ())),
                                  preferred_element_type=jnp.float32)
        denom_n = jnp.dot(oh, denom_b, preferred_element_type=jnp.float32)
        a = a_un / jnp.where(denom_n > 0, denom_n, 1.0)
        r_vec = lax.dot_general(oh, a * h_, (((0,), (0,)), ((), ())),
                                preferred_element_type=jnp.float32)
        o_ref[...] = (
            jnp.dot(q_row, wq_ref[...], preferred_element_type=jnp.float32)
            + jnp.dot(r_vec, wr_ref[...], preferred_element_type=jnp.float32)
            + bo_ref[...]
        )

    return pl.pallas_call(
        body,
        out_shape=jax.ShapeDtypeStruct((B, OUT_DIM), jnp.float32),
    )(h, batch_col, bihs_r, bhhs_r, WqT, WrT, bout_r)


_R_EXPAND = np.repeat(np.eye(DIM, dtype=np.float32), DIM, axis=1)
_S_FOLD = (np.arange(128)[:, None] % DIM == np.arange(DIM)[None, :]
           ).astype(np.float32)


def kernel(x, edge_index, edge_attr, batch, W0, b0, Wn1, bn1, Wn2, bn2,
           Wroot, bconv, Wih, Whh, bih, bhh, Wih_s, Whh_s, bih_s, bhh_s,
           Wout, bout):
    src = edge_index[0]
    dst = edge_index[1]
    pad_idx = jnp.full((EP - E,), N, jnp.int32)
    srcp = jnp.concatenate([src, pad_idx])
    dstp = jnp.concatenate([dst, pad_idx])
    dst3 = dstp.reshape(NW, NCHUNK, CHUNK)
    ep2 = EP // 2
    dst3A = dstp[:ep2].reshape(NW, NCHUNK // 2, CHUNK)
    dst3B = dstp[ep2:].reshape(NW, NCHUNK // 2, CHUNK)

    xp = jnp.pad(x, ((0, NP - N), (0, 0)))
    eap = jnp.pad(edge_attr, ((0, EP - E), (0, 0)))
    batch_col = jnp.pad(batch, (0, NP - N), constant_values=B).reshape(NP, 1)

    Wn2T = Wn2.T
    Bn2 = bn2.reshape(DIM, DIM)
    Rm = jnp.asarray(_R_EXPAND)
    Sm = jnp.asarray(_S_FOLD)
    zrow = jnp.zeros((SP, DIM), jnp.float32)
    onesE = jnp.ones((EP, DIM), jnp.float32)

    h0 = _row_matmul_act(xp, W0.T, b0.reshape(1, DIM), N, BLK_N, True)
    hid = _row_matmul_act(eap, Wn1.T, bn1.reshape(1, DIM), EP, BLK_E, True)

    degP = _sc_scatter_add(onesE, dst3, zrow)
    inv_deg = _deg_finalize(degP)

    WihT = Wih.T
    WhhT = Whh.T
    bconv_r = bconv.reshape(1, DIM)
    bih_r = bih.reshape(1, 3 * DIM)
    bhh_r = bhh.reshape(1, 3 * DIM)

    def step(_, h):
        xj = _sc_gather(h, srcp, 0, EP)
        msgA = _msg_bilinear(xj, hid, Rm, Wn2T, Sm, Bn2, 0)
        aggA = _sc_scatter_add(msgA, dst3A, zrow)
        msgB = _msg_bilinear(xj, hid, Rm, Wn2T, Sm, Bn2, ep2 // BLK_E)
        aggB = _sc_scatter_add(msgB, dst3B, zrow)
        return _node_update(aggA, aggB, inv_deg, h, Wroot, WihT, WhhT,
                            bconv_r, bih_r, bhh_r)

    h = lax.fori_loop(0, 6, step, h0)

    WoutT = Wout.T
    return _set2set_out(h, batch_col,
                        bih_s.reshape(1, 4 * DIM), bhh_s.reshape(1, 4 * DIM),
                        WoutT[:DIM], WoutT[DIM:], bout.reshape(1, OUT_DIM))


# speedup vs baseline: 2.8196x; 1.0124x over previous
import functools

import jax
import jax.numpy as jnp
import numpy as np
from jax import lax
from jax.experimental import pallas as pl
from jax.experimental.pallas import tpu as pltpu
from jax.experimental.pallas import tpu_sc as plsc

N = 10000
E = 160000
F_IN = 14
DIM = 32
B = 64
OUT_DIM = 16

NC = 2
NS = 16
NW = NC * NS
NP = 10240
EP = 163840
EPW = EP // NW
CHUNK = 128
NCHUNK = EPW // CHUNK
SP = NP // NS

BLK_E = 2048
BLK_N = 1024

_SC_MESH = dict(core_axis_name="c", subcore_axis_name="s")
_SC_PARAMS = pltpu.CompilerParams(use_tc_tiling_on_sc=False)


def _leaky(v):
    return jnp.where(v >= 0, v, 0.01 * v)


_GNB = 4


def _sc_gather(table, idx, half_off, n_edges):
    epw = n_edges // NW
    nchunk = epw // CHUNK

    @functools.partial(
        pl.kernel,
        out_type=jax.ShapeDtypeStruct((n_edges, DIM), jnp.float32),
        mesh=plsc.VectorSubcoreMesh(**_SC_MESH),
        compiler_params=_SC_PARAMS,
        scratch_types=[
            pltpu.VMEM((epw,), jnp.int32),
            pltpu.VMEM((_GNB, CHUNK, DIM), jnp.float32),
            pltpu.VMEM_SHARED((NP, DIM), jnp.float32),
            pltpu.SemaphoreType.DMA((_GNB,)),
            pltpu.SemaphoreType.DMA((_GNB,)),
        ],
    )
    def k(table_hbm, idx_hbm, out_hbm, idx_v, rows_v, stable, gsem, osem):
        sid = lax.axis_index("s")
        wid = sid * NC + lax.axis_index("c")
        base = wid * epw
        pltpu.sync_copy(table_hbm.at[pl.ds(sid * SP, SP)],
                        stable.at[pl.ds(sid * SP, SP)])
        pltpu.sync_copy(idx_hbm.at[pl.ds(half_off + base, epw)], idx_v)
        plsc.subcore_barrier()

        def start(j):
            return pltpu.async_copy(
                stable.at[idx_v.at[pl.ds(j * CHUNK, CHUNK)]],
                rows_v.at[j % _GNB], gsem.at[j % _GNB],
            )

        gd = [None] * _GNB
        od = [None] * _GNB
        for j in range(min(_GNB - 1, nchunk)):
            gd[j % _GNB] = start(j)
        for j in range(nchunk):
            b = j % _GNB
            gd[b].wait()
            gd[b] = None
            od[b] = pltpu.async_copy(
                rows_v.at[b], out_hbm.at[pl.ds(base + j * CHUNK, CHUNK)],
                osem.at[b],
            )
            nj = j + _GNB - 1
            if nj < nchunk:
                nb = nj % _GNB
                if od[nb] is not None:
                    od[nb].wait()
                    od[nb] = None
                gd[nb] = start(nj)
        for b in range(_GNB):
            if od[b] is not None:
                od[b].wait()

    return k(table, idx)


_SNB = 4


def _sc_scatter_add(msg, dst3, zrow):
    n_edges = msg.shape[0]
    epw = n_edges // NW
    nchunk = epw // CHUNK

    @functools.partial(
        pl.kernel,
        out_type=jax.ShapeDtypeStruct((NC, NP, DIM), jnp.float32),
        mesh=plsc.VectorSubcoreMesh(**_SC_MESH),
        compiler_params=_SC_PARAMS,
        scratch_types=[
            pltpu.VMEM((nchunk, CHUNK), jnp.int32),
            pltpu.VMEM((_SNB, CHUNK, DIM), jnp.float32),
            pltpu.VMEM_SHARED((NP, DIM), jnp.float32),
            pltpu.SemaphoreType.DMA((_SNB,)),
            pltpu.SemaphoreType.DMA((_SNB,)),
        ],
    )
    def k(msg_hbm, dst_hbm, z_hbm, out_hbm, idx_v, mbuf, acc, msem, asem):
        cid = lax.axis_index("c")
        sid = lax.axis_index("s")
        wid = sid * NC + cid
        pltpu.sync_copy(z_hbm, acc.at[pl.ds(sid * SP, SP)])
        pltpu.sync_copy(dst_hbm.at[wid], idx_v)
        plsc.subcore_barrier()
        base = wid * epw

        def load(j):
            return pltpu.async_copy(
                msg_hbm.at[pl.ds(base + j * CHUNK, CHUNK)],
                mbuf.at[j % _SNB], msem.at[j % _SNB],
            )

        md = [None] * _SNB
        ad = [None] * _SNB
        for j in range(min(_SNB - 1, nchunk)):
            md[j % _SNB] = load(j)
        for j in range(nchunk):
            b = j % _SNB
            md[b].wait()
            md[b] = None
            ad[b] = pltpu.async_copy(
                mbuf.at[b], acc.at[idx_v.at[j]], asem.at[b], add=True
            )
            nj = j + _SNB - 1
            if nj < nchunk:
                nb = nj % _SNB
                if ad[nb] is not None:
                    ad[nb].wait()
                    ad[nb] = None
                md[nb] = load(nj)
        for b in range(_SNB):
            if ad[b] is not None:
                ad[b].wait()
        plsc.subcore_barrier()
        pltpu.sync_copy(
            acc.at[pl.ds(sid * SP, SP)], out_hbm.at[cid, pl.ds(sid * SP, SP)]
        )

    return k(msg, dst3, zrow)


def _row_matmul_act(xp, WT, brow, n_valid, blk, act):
    rows_total, f_in = xp.shape
    f_out = WT.shape[1]

    def body(x_ref, w_ref, b_ref, o_ref):
        i = pl.program_id(0)
        v = jnp.dot(x_ref[...], w_ref[...], preferred_element_type=jnp.float32)
        v = v + b_ref[...]
        if act:
            v = _leaky(v)
        rows = i * blk + lax.broadcasted_iota(jnp.int32, (blk, 1), 0)
        o_ref[...] = jnp.where(rows < n_valid, v, 0.0)

    return pl.pallas_call(
        body,
        grid=(rows_total // blk,),
        in_specs=[
            pl.BlockSpec((blk, f_in), lambda i: (i, 0)),
            pl.BlockSpec((f_in, f_out), lambda i: (0, 0)),
            pl.BlockSpec((1, f_out), lambda i: (0, 0)),
        ],
        out_specs=pl.BlockSpec((blk, f_out), lambda i: (i, 0)),
        out_shape=jax.ShapeDtypeStruct((rows_total, f_out), jnp.float32),
    )(xp, WT, brow)


def _msg_bilinear(xj, hid, Rm, Wn2T, Sm, Bn2, hoff=0):
    def body(x_ref, h_ref, r_ref, w_ref, s_ref, b_ref, o_ref):
        xb = x_ref[...]
        y = jnp.dot(h_ref[...].astype(jnp.bfloat16), w_ref[...],
                    preferred_element_type=jnp.float32).astype(jnp.bfloat16)
        z = jnp.dot(xb.astype(jnp.bfloat16), r_ref[...],
                    preferred_element_type=jnp.float32).astype(jnp.bfloat16) * y
        for wdt in (512, 256, 128):
            z = z[:, :wdt] + z[:, wdt:2 * wdt]
        o_ref[...] = (
            jnp.dot(z, s_ref[...], preferred_element_type=jnp.float32)
            + jnp.dot(xb, b_ref[...], preferred_element_type=jnp.float32)
        )

    d2 = DIM * DIM
    n_edges = EP // 2
    return pl.pallas_call(
        body,
        grid=(n_edges // BLK_E,),
        in_specs=[
            pl.BlockSpec((BLK_E, DIM), lambda i: (i + hoff, 0)),
            pl.BlockSpec((BLK_E, DIM), lambda i: (i + hoff, 0)),
            pl.BlockSpec((DIM, d2), lambda i: (0, 0)),
            pl.BlockSpec((DIM, d2), lambda i: (0, 0)),
            pl.BlockSpec((128, DIM), lambda i: (0, 0)),
            pl.BlockSpec((DIM, DIM), lambda i: (0, 0)),
        ],
        out_specs=pl.BlockSpec((BLK_E, DIM), lambda i: (i, 0)),
        out_shape=jax.ShapeDtypeStruct((n_edges, DIM), jnp.float32),
    )(xj, hid, Rm, Wn2T, Sm, Bn2)


def _deg_finalize(degP):
    def body(p_ref, o_ref):
        deg = p_ref[0, :, 0:1] + p_ref[1, :, 0:1]
        o_ref[...] = 1.0 / jnp.maximum(deg, 1.0)

    return pl.pallas_call(
        body,
        grid=(NP // BLK_N,),
        in_specs=[pl.BlockSpec((NC, BLK_N, DIM), lambda i: (0, i, 0))],
        out_specs=pl.BlockSpec((BLK_N, 1), lambda i: (i, 0)),
        out_shape=jax.ShapeDtypeStruct((NP, 1), jnp.float32),
    )(degP)


def _node_update(aggP, aggQ, inv_deg, h, Wroot, WihT, WhhT, bconv_r, bih_r,
                 bhh_r):
    def body(p_ref, q_ref, iv_ref, h_ref, wr_ref, wi_ref, wh_ref, bc_ref,
             bi_ref, bh_ref, o_ref):
        i = pl.program_id(0)
        h_ = h_ref[...]
        agg = (p_ref[0] + p_ref[1] + q_ref[0] + q_ref[1]) * iv_ref[...]
        m = _leaky(
            agg
            + jnp.dot(h_, wr_ref[...], preferred_element_type=jnp.float32)
            + bc_ref[...]
        )
        gi = jnp.dot(m, wi_ref[...], preferred_element_type=jnp.float32) + bi_ref[...]
        gh = jnp.dot(h_, wh_ref[...], preferred_element_type=jnp.float32) + bh_ref[...]
        r = jax.nn.sigmoid(gi[:, 0:DIM] + gh[:, 0:DIM])
        z = jax.nn.sigmoid(gi[:, DIM:2 * DIM] + gh[:, DIM:2 * DIM])
        n = jnp.tanh(gi[:, 2 * DIM:3 * DIM] + r * gh[:, 2 * DIM:3 * DIM])
        hn = (1.0 - z) * n + z * h_
        rows = i * BLK_N + lax.broadcasted_iota(jnp.int32, (BLK_N, 1), 0)
        o_ref[...] = jnp.where(rows < N, hn, 0.0)

    g3 = 3 * DIM
    return pl.pallas_call(
        body,
        grid=(NP // BLK_N,),
        in_specs=[
            pl.BlockSpec((NC, BLK_N, DIM), lambda i: (0, i, 0)),
            pl.BlockSpec((NC, BLK_N, DIM), lambda i: (0, i, 0)),
            pl.BlockSpec((BLK_N, 1), lambda i: (i, 0)),
            pl.BlockSpec((BLK_N, DIM), lambda i: (i, 0)),
            pl.BlockSpec((DIM, DIM), lambda i: (0, 0)),
            pl.BlockSpec((DIM, g3), lambda i: (0, 0)),
            pl.BlockSpec((DIM, g3), lambda i: (0, 0)),
            pl.BlockSpec((1, DIM), lambda i: (0, 0)),
            pl.BlockSpec((1, g3), lambda i: (0, 0)),
            pl.BlockSpec((1, g3), lambda i: (0, 0)),
        ],
        out_specs=pl.BlockSpec((BLK_N, DIM), lambda i: (i, 0)),
        out_shape=jax.ShapeDtypeStruct((NP, DIM), jnp.float32),
    )(aggP, aggQ, inv_deg, h, Wroot, WihT, WhhT, bconv_r, bih_r, bhh_r)


def _set2set_out(h, batch_col, bihs_r, bhhs_r, WqT, WrT, bout_r):
    def body(h_ref, b_ref, bi_ref, bh_ref, wq_ref, wr_ref, bo_ref, o_ref):
        h_ = h_ref[...]
        bb = b_ref[...]
        g = bi_ref[...] + bh_ref[...]
        ig = jax.nn.sigmoid(g[:, 0:DIM])
        fg = jax.nn.sigmoid(g[:, DIM:2 * DIM])
        gg = jnp.tanh(g[:, 2 * DIM:3 * DIM])
        og = jax.nn.sigmoid(g[:, 3 * DIM:4 * DIM])
        cs = ig * gg + fg * 0.0
        q_row = og * jnp.tanh(cs)
        e = jnp.sum(h_ * q_row, axis=1, keepdims=True)
        ids = lax.broadcasted_iota(jnp.int32, (NP, B), 1)
        oh = (bb == ids).astype(jnp.float32)
        neg = jnp.float32(-1e30)
        emax_b = jnp.max(jnp.where(oh > 0, e, neg), axis=0, keepdims=True)
        emax_b = jnp.where(emax_b > neg * 0.5, emax_b, 0.0)
        emax_n = jnp.sum(oh * emax_b, axis=1, keepdims=True)
        valid = jnp.sum(oh, axis=1, keepdims=True)
        a_un = jnp.exp(e - emax_n) * valid
        denom_b = lax.dot_general(oh, a_un, (((0,), (0,)), ((), ())),
                                  preferred_element_type=jnp.float32)
        denom_n = jnp.dot(oh, denom_b, preferred_element_type=jnp.float32)
        a = a_un / jnp.where(denom_n > 0, denom_n, 1.0)
        r_vec = lax.dot_general(oh, a * h_, (((0,), (0,)), ((), ())),
                                preferred_element_type=jnp.float32)
        o_ref[...] = (
            jnp.dot(q_row, wq_ref[...], preferred_element_type=jnp.float32)
            + jnp.dot(r_vec, wr_ref[...], preferred_element_type=jnp.float32)
            + bo_ref[...]
        )

    return pl.pallas_call(
        body,
        out_shape=jax.ShapeDtypeStruct((B, OUT_DIM), jnp.float32),
    )(h, batch_col, bihs_r, bhhs_r, WqT, WrT, bout_r)


_R_EXPAND = np.repeat(np.eye(DIM, dtype=np.float32), DIM, axis=1)
_S_FOLD = (np.arange(128)[:, None] % DIM == np.arange(DIM)[None, :]
           ).astype(np.float32)


def kernel(x, edge_index, edge_attr, batch, W0, b0, Wn1, bn1, Wn2, bn2,
           Wroot, bconv, Wih, Whh, bih, bhh, Wih_s, Whh_s, bih_s, bhh_s,
           Wout, bout):
    src = edge_index[0]
    dst = edge_index[1]
    pad_idx = jnp.full((EP - E,), N, jnp.int32)
    srcp = jnp.concatenate([src, pad_idx])
    dstp = jnp.concatenate([dst, pad_idx])
    dst3 = dstp.reshape(NW, NCHUNK, CHUNK)
    ep2 = EP // 2
    dst3A = dstp[:ep2].reshape(NW, NCHUNK // 2, CHUNK)
    dst3B = dstp[ep2:].reshape(NW, NCHUNK // 2, CHUNK)

    xp = jnp.pad(x, ((0, NP - N), (0, 0)))
    eap = jnp.pad(edge_attr, ((0, EP - E), (0, 0)))
    batch_col = jnp.pad(batch, (0, NP - N), constant_values=B).reshape(NP, 1)

    Wn2T = Wn2.T.astype(jnp.bfloat16)
    Bn2 = bn2.reshape(DIM, DIM)
    Rm = jnp.asarray(_R_EXPAND).astype(jnp.bfloat16)
    Sm = jnp.asarray(_S_FOLD).astype(jnp.bfloat16)
    zrow = jnp.zeros((SP, DIM), jnp.float32)
    onesE = jnp.ones((EP, DIM), jnp.float32)

    h0 = _row_matmul_act(xp, W0.T, b0.reshape(1, DIM), N, BLK_N, True)
    hid = _row_matmul_act(eap, Wn1.T, bn1.reshape(1, DIM), EP, BLK_E, True)

    degP = _sc_scatter_add(onesE, dst3, zrow)
    inv_deg = _deg_finalize(degP)

    WihT = Wih.T
    WhhT = Whh.T
    bconv_r = bconv.reshape(1, DIM)
    bih_r = bih.reshape(1, 3 * DIM)
    bhh_r = bhh.reshape(1, 3 * DIM)

    def step(_, h):
        xj = _sc_gather(h, srcp, 0, EP)
        msgA = _msg_bilinear(xj, hid, Rm, Wn2T, Sm, Bn2, 0)
        aggA = _sc_scatter_add(msgA, dst3A, zrow)
        msgB = _msg_bilinear(xj, hid, Rm, Wn2T, Sm, Bn2, ep2 // BLK_E)
        aggB = _sc_scatter_add(msgB, dst3B, zrow)
        return _node_update(aggA, aggB, inv_deg, h, Wroot, WihT, WhhT,
                            bconv_r, bih_r, bhh_r)

    h = lax.fori_loop(0, 6, step, h0)

    WoutT = Wout.T
    return _set2set_out(h, batch_col,
                        bih_s.reshape(1, 4 * DIM), bhh_s.reshape(1, 4 * DIM),
                        WoutT[:DIM], WoutT[DIM:], bout.reshape(1, OUT_DIM))
